# baseline (speedup 1.0000x reference)
import functools

import jax
import jax.numpy as jnp
from jax import lax
from jax.experimental import pallas as pl
from jax.experimental.pallas import tpu as pltpu
from jax.experimental.pallas import tpu_sc as plsc

_VOCAB = 1000
_K = 8
_EMBED = 32
_B = 4096
_HID = 512
_OUT = 1000

_NC = 2
_NS = 16
_NW = _NC * _NS

_SC_S = 2048
_SC_R0 = _B - _SC_S
_SC_RPW = _SC_S // _NW
_SC_SLAB = 4
_SC_NSLAB = _SC_RPW // _SC_SLAB

_ROW_W = _K * _VOCAB
_NVREG = 63

_BB = 512


_HALF = _VOCAB // 2
_UNROLL = 5


def _slab_argmax(buf, g, lanei):
    rvec = g * 2 + lax.shift_right_logical(lanei, 3)
    kvec = lanei - lax.shift_right_logical(lanei, 3) * _K
    colbase = kvec * _VOCAB
    neg = jnp.full((16,), -jnp.inf, dtype=jnp.float32)
    zero = jnp.zeros((16,), dtype=jnp.int32)

    def step(carry, off):
        ma, pa, mb, pb, ca, cb = carry
        va = plsc.load_gather(buf, [rvec, ca])
        vb = plsc.load_gather(buf, [rvec, cb])
        ua = va > ma
        ub = vb > mb
        ma = jnp.where(ua, va, ma)
        pa = jnp.where(ua, ca, pa)
        mb = jnp.where(ub, vb, mb)
        pb = jnp.where(ub, cb, pb)
        return (ma, pa, mb, pb, ca + 1, cb + 1)

    init = (neg, zero, neg, zero, colbase, colbase + _HALF)

    def body(i, carry):
        for u in range(_UNROLL):
            carry = step(carry, u)
        return carry

    ma, pa, mb, pb, _, _ = lax.fori_loop(0, _HALF // _UNROLL, body, init)
    takeb = mb > ma
    p = jnp.where(takeb, pb, pa)
    return p - colbase


@functools.lru_cache(maxsize=1)
def _make_sc_argmax():
    mesh = plsc.VectorSubcoreMesh(core_axis_name="c", subcore_axis_name="s")

    @functools.partial(
        pl.kernel,
        mesh=mesh,
        out_type=jax.ShapeDtypeStruct((_NW, _SC_RPW * _K), jnp.int32),
        scratch_types=[
            pltpu.VMEM((_SC_SLAB, _ROW_W), jnp.float32),
            pltpu.VMEM((_SC_SLAB, _ROW_W), jnp.float32),
            pltpu.VMEM((_SC_RPW * _K,), jnp.int32),
            pltpu.SemaphoreType.DMA,
            pltpu.SemaphoreType.DMA,
        ],
        compiler_params=pltpu.CompilerParams(use_tc_tiling_on_sc=True, needs_layout_passes=False),
    )
    def sc_argmax(x_hbm, idx_hbm, buf0, buf1, idx_v, sem0, sem1):
        wid = lax.axis_index("s") * _NC + lax.axis_index("c")
        base = _SC_R0 + wid * _SC_RPW
        lanei = lax.broadcasted_iota(jnp.int32, (16,), 0)
        bufs = (buf0, buf1)
        sems = (sem0, sem1)
        cps = [None, None]
        cps[0] = pltpu.async_copy(
            x_hbm.at[pl.ds(base, _SC_SLAB)], buf0, sem0)
        for s in range(_SC_NSLAB):
            cur = s % 2
            if s + 1 < _SC_NSLAB:
                cps[1 - cur] = pltpu.async_copy(
                    x_hbm.at[pl.ds(base + (s + 1) * _SC_SLAB, _SC_SLAB)],
                    bufs[1 - cur], sems[1 - cur])
            cps[cur].wait()
            buf = bufs[cur]
            for g in range(_SC_SLAB * _K // 16):
                idx_v[pl.ds((s * _SC_SLAB * _K // 16 + g) * 16, 16)] = (
                    _slab_argmax(buf, g, lanei))
        pltpu.sync_copy(idx_v, idx_hbm.at[wid])

    return sc_argmax


def _onehot_mlp(xk_or_idx, emb, w1_ref, b1_ref, w2_ref, b2_ref, o_ref,
                fe_parts):
    fe = jnp.concatenate(fe_parts, axis=1)
    h = jnp.dot(fe, w1_ref[...], preferred_element_type=jnp.float32)
    h = jnp.maximum(h + b1_ref[...], 0.0)
    o = jnp.dot(h, w2_ref[...], preferred_element_type=jnp.float32)
    o_ref[...] = o + b2_ref[...]


def _fused_body(x_ref, emb_ref, w1_ref, b1_ref, w2_ref, b2_ref, o_ref):
    xb = x_ref[...]
    emb = emb_ref[...]
    fe_parts = []
    for k in range(_K):
        xk = xb[:, k * _VOCAB:(k + 1) * _VOCAB]
        m = jnp.max(xk, axis=1, keepdims=True)
        ii = lax.broadcasted_iota(jnp.int32, xk.shape, 1)
        cand = jnp.where(xk == m, ii, _VOCAB)
        idxk = jnp.min(cand, axis=1, keepdims=True)
        onehot = (ii == idxk).astype(jnp.float32)
        fe_parts.append(
            jnp.dot(onehot, emb, preferred_element_type=jnp.float32))
    _onehot_mlp(None, None, w1_ref, b1_ref, w2_ref, b2_ref, o_ref, fe_parts)


def _fused(x, emb, w1, b1, w2, b2, nrows):
    grid = nrows // _BB
    return pl.pallas_call(
        _fused_body,
        grid=(grid,),
        in_specs=[
            pl.BlockSpec((_BB, _ROW_W), lambda i: (i, 0)),
            pl.BlockSpec((_VOCAB, _EMBED), lambda i: (0, 0)),
            pl.BlockSpec((_K * _EMBED, _HID), lambda i: (0, 0)),
            pl.BlockSpec((1, _HID), lambda i: (0, 0)),
            pl.BlockSpec((_HID, _OUT), lambda i: (0, 0)),
            pl.BlockSpec((1, _OUT), lambda i: (0, 0)),
        ],
        out_specs=pl.BlockSpec((_BB, _OUT), lambda i: (i, 0)),
        out_shape=jax.ShapeDtypeStruct((_B, _OUT), jnp.float32),
    )(x, emb, w1, b1, w2, b2)


def _tail_body(prev_ref, idx_ref, emb_ref, w1_ref, b1_ref, w2_ref, b2_ref,
               o_ref):
    del prev_ref
    idxb = idx_ref[...]
    emb = emb_ref[...]
    fe_parts = []
    for k in range(_K):
        idxk = idxb[:, k:k + 1]
        ii = lax.broadcasted_iota(jnp.int32, (idxb.shape[0], _VOCAB), 1)
        onehot = (ii == idxk).astype(jnp.float32)
        fe_parts.append(
            jnp.dot(onehot, emb, preferred_element_type=jnp.float32))
    _onehot_mlp(None, None, w1_ref, b1_ref, w2_ref, b2_ref, o_ref, fe_parts)


def _tail_mlp(prev_out, idx_sc, emb, w1, b1, w2, b2):
    grid = _SC_S // _BB
    off = _SC_R0 // _BB
    return pl.pallas_call(
        _tail_body,
        grid=(grid,),
        in_specs=[
            pl.BlockSpec(memory_space=pl.ANY),
            pl.BlockSpec((_BB, _K), lambda i: (i, 0)),
            pl.BlockSpec((_VOCAB, _EMBED), lambda i: (0, 0)),
            pl.BlockSpec((_K * _EMBED, _HID), lambda i: (0, 0)),
            pl.BlockSpec((1, _HID), lambda i: (0, 0)),
            pl.BlockSpec((_HID, _OUT), lambda i: (0, 0)),
            pl.BlockSpec((1, _OUT), lambda i: (0, 0)),
        ],
        out_specs=pl.BlockSpec((_BB, _OUT), lambda i: (off + i, 0)),
        out_shape=jax.ShapeDtypeStruct((_B, _OUT), jnp.float32),
        input_output_aliases={0: 0},
    )(prev_out, idx_sc, emb, w1, b1, w2, b2)


def kernel(x, emb, W1, b1, W2, b2):
    b1r = b1.reshape(1, _HID)
    b2r = b2.reshape(1, _OUT)
    idx_sc = _make_sc_argmax()(x).reshape(_SC_S, _K)
    head = _fused(x, emb, W1, b1r, W2, b2r, _SC_R0)
    return _tail_mlp(head, idx_sc, emb, W1, b1r, W2, b2r)


# speedup vs baseline: 4.1384x; 4.1384x over previous
import functools

import jax
import jax.numpy as jnp
from jax import lax
from jax.experimental import pallas as pl
from jax.experimental.pallas import tpu as pltpu
from jax.experimental.pallas import tpu_sc as plsc

_VOCAB = 1000
_K = 8
_EMBED = 32
_B = 4096
_HID = 512
_OUT = 1000

_NC = 2
_NS = 16
_NW = _NC * _NS

_SC_S = 2048
_SC_R0 = _B - _SC_S
_SC_RPW = _SC_S // _NW
_SC_SLAB = 4
_SC_NSLAB = _SC_RPW // _SC_SLAB

_ROW_W = _K * _VOCAB
_NVREG = 63

_BB = 512


_HALF = _VOCAB // 2
_UNROLL = 5


def _slab_argmax(buf, g, lanei):
    rvec = g * 2 + lax.shift_right_logical(lanei, 3)
    kvec = lanei - lax.shift_right_logical(lanei, 3) * _K
    colbase = kvec * _VOCAB
    neg = jnp.full((16,), -jnp.inf, dtype=jnp.float32)
    zero = jnp.zeros((16,), dtype=jnp.int32)

    def step(carry, off):
        ma, pa, mb, pb, ca, cb = carry
        va = plsc.load_gather(buf, [rvec, ca])
        vb = plsc.load_gather(buf, [rvec, cb])
        ua = va > ma
        ub = vb > mb
        ma = jnp.where(ua, va, ma)
        pa = jnp.where(ua, ca, pa)
        mb = jnp.where(ub, vb, mb)
        pb = jnp.where(ub, cb, pb)
        return (ma, pa, mb, pb, ca + 1, cb + 1)

    init = (neg, zero, neg, zero, colbase, colbase + _HALF)

    def body(i, carry):
        for u in range(_UNROLL):
            carry = step(carry, u)
        return carry

    ma, pa, mb, pb, _, _ = lax.fori_loop(0, _HALF // _UNROLL, body, init)
    takeb = mb > ma
    p = jnp.where(takeb, pb, pa)
    return p - colbase


@functools.lru_cache(maxsize=1)
def _make_sc_argmax():
    mesh = plsc.VectorSubcoreMesh(core_axis_name="c", subcore_axis_name="s")

    @functools.partial(
        pl.kernel,
        mesh=mesh,
        out_type=jax.ShapeDtypeStruct((_NW, _SC_RPW * _K), jnp.int32),
        scratch_types=[
            pltpu.VMEM((_SC_SLAB, _ROW_W), jnp.float32),
            pltpu.VMEM((_SC_SLAB, _ROW_W), jnp.float32),
            pltpu.VMEM((_SC_RPW * _K,), jnp.int32),
            pltpu.SemaphoreType.DMA,
            pltpu.SemaphoreType.DMA,
        ],
        compiler_params=pltpu.CompilerParams(use_tc_tiling_on_sc=True, needs_layout_passes=False),
    )
    def sc_argmax(x_hbm, idx_hbm, buf0, buf1, idx_v, sem0, sem1):
        wid = lax.axis_index("s") * _NC + lax.axis_index("c")
        base = _SC_R0 + wid * _SC_RPW
        lanei = lax.broadcasted_iota(jnp.int32, (16,), 0)
        bufs = (buf0, buf1)
        sems = (sem0, sem1)
        cps = [None, None]
        cps[0] = pltpu.async_copy(
            x_hbm.at[pl.ds(base, _SC_SLAB)], buf0, sem0)
        for s in range(_SC_NSLAB):
            cur = s % 2
            if s + 1 < _SC_NSLAB:
                cps[1 - cur] = pltpu.async_copy(
                    x_hbm.at[pl.ds(base + (s + 1) * _SC_SLAB, _SC_SLAB)],
                    bufs[1 - cur], sems[1 - cur])
            cps[cur].wait()
            buf = bufs[cur]
            for g in range(_SC_SLAB * _K // 16):
                idx_v[pl.ds((s * _SC_SLAB * _K // 16 + g) * 16, 16)] = (
                    _slab_argmax(buf, g, lanei))
        pltpu.sync_copy(idx_v, idx_hbm.at[wid])

    return sc_argmax


def _mlp_t(fe_t, w1_ref, b1_ref, w2t_ref, b2_ref, o_ref):
    h = lax.dot_general(w1_ref[...], fe_t, (((0,), (0,)), ((), ())),
                        preferred_element_type=jnp.float32)
    h = jnp.maximum(h + b1_ref[...], 0.0)
    o = jnp.dot(w2t_ref[...], h, preferred_element_type=jnp.float32)
    o_ref[...] = o + b2_ref[...]


def _fused_body(xt_ref, embt_ref, w1_ref, b1_ref, w2t_ref, b2_ref, o_ref):
    xb = xt_ref[...]
    embt = embt_ref[...]
    fe_parts = []
    for k in range(_K):
        xk = xb[k * _VOCAB:(k + 1) * _VOCAB, :]
        m = jnp.max(xk, axis=0, keepdims=True)
        ii = lax.broadcasted_iota(jnp.int32, xk.shape, 0)
        cand = jnp.where(xk == m, ii, _VOCAB)
        idxk = jnp.min(cand, axis=0, keepdims=True)
        onehot = (ii == idxk).astype(jnp.float32)
        fe_parts.append(
            jnp.dot(embt, onehot, preferred_element_type=jnp.float32))
    fe_t = jnp.concatenate(fe_parts, axis=0)
    _mlp_t(fe_t, w1_ref, b1_ref, w2t_ref, b2_ref, o_ref)


def _fused(xt, embt, w1, b1c, w2t, b2c, ncols):
    grid = ncols // _BB
    return pl.pallas_call(
        _fused_body,
        grid=(grid,),
        in_specs=[
            pl.BlockSpec((_ROW_W, _BB), lambda i: (0, i)),
            pl.BlockSpec((_EMBED, _VOCAB), lambda i: (0, 0)),
            pl.BlockSpec((_K * _EMBED, _HID), lambda i: (0, 0)),
            pl.BlockSpec((_HID, 1), lambda i: (0, 0)),
            pl.BlockSpec((_OUT, _HID), lambda i: (0, 0)),
            pl.BlockSpec((_OUT, 1), lambda i: (0, 0)),
        ],
        out_specs=pl.BlockSpec((_OUT, _BB), lambda i: (0, i)),
        out_shape=jax.ShapeDtypeStruct((_OUT, _B), jnp.float32),
    )(xt, embt, w1, b1c, w2t, b2c)


def _tail_body(prev_ref, idx_ref, emb_ref, w1_ref, b1_ref, w2_ref, b2_ref,
               o_ref):
    del prev_ref
    idxb = idx_ref[...]
    emb = emb_ref[...]
    fe_parts = []
    for k in range(_K):
        idxk = idxb[:, k:k + 1]
        ii = lax.broadcasted_iota(jnp.int32, (idxb.shape[0], _VOCAB), 1)
        onehot = (ii == idxk).astype(jnp.float32)
        fe_parts.append(
            jnp.dot(onehot, emb, preferred_element_type=jnp.float32))
    _onehot_mlp(None, None, w1_ref, b1_ref, w2_ref, b2_ref, o_ref, fe_parts)


def _tail_mlp(prev_out, idx_sc, emb, w1, b1, w2, b2):
    grid = _SC_S // _BB
    off = _SC_R0 // _BB
    return pl.pallas_call(
        _tail_body,
        grid=(grid,),
        in_specs=[
            pl.BlockSpec(memory_space=pl.ANY),
            pl.BlockSpec((_BB, _K), lambda i: (i, 0)),
            pl.BlockSpec((_VOCAB, _EMBED), lambda i: (0, 0)),
            pl.BlockSpec((_K * _EMBED, _HID), lambda i: (0, 0)),
            pl.BlockSpec((1, _HID), lambda i: (0, 0)),
            pl.BlockSpec((_HID, _OUT), lambda i: (0, 0)),
            pl.BlockSpec((1, _OUT), lambda i: (0, 0)),
        ],
        out_specs=pl.BlockSpec((_BB, _OUT), lambda i: (off + i, 0)),
        out_shape=jax.ShapeDtypeStruct((_B, _OUT), jnp.float32),
        input_output_aliases={0: 0},
    )(prev_out, idx_sc, emb, w1, b1, w2, b2)


def kernel(x, emb, W1, b1, W2, b2):
    xt = x.T
    embt = emb.T
    w2t = W2.T
    b1c = b1.reshape(_HID, 1)
    b2c = b2.reshape(_OUT, 1)
    out_t = _fused(xt, embt, W1, b1c, w2t, b2c, _B)
    return out_t.T
